# speedup vs baseline: 3.5528x; 3.5528x over previous
import jax
import jax.numpy as jnp
from jax import lax
from jax.experimental import pallas as pl
from jax.experimental.pallas import tpu as pltpu
from jax.experimental.pallas import tpu_sc as plsc

N_NODES = 10000
N_EDGES = 320000
DIM = 128

NC = 2
NS = 16
NW = NC * NS

N_PAD = 10240
EC = 128
CHUNKS_PER_W = 79
E_PAD = NW * CHUNKS_PER_W * EC
ROWS_PER_SUB = N_PAD // NS


def _proj_body(x_ref, we_ref, be_ref, ps_ref, pd_ref):
    xv = x_ref[...]
    ps_ref[...] = jnp.dot(xv, we_ref[0:DIM, :], preferred_element_type=jnp.float32)
    pd_ref[...] = (
        jnp.dot(xv, we_ref[DIM : 2 * DIM, :], preferred_element_type=jnp.float32)
        + be_ref[...]
    )


def _project(x_pad, W_edge, b_edge):
    return pl.pallas_call(
        _proj_body,
        out_shape=[
            jax.ShapeDtypeStruct((N_PAD, DIM), jnp.float32),
            jax.ShapeDtypeStruct((N_PAD, DIM), jnp.float32),
        ],
    )(x_pad, W_edge, b_edge.reshape(1, DIM))


def _sc_body(ps_hbm, pd_hbm, src_hbm, dst_hbm, out_hbm,
             idx_s, idx_d, rows_a, rows_b, agg, sem):
    cid = lax.axis_index("c")
    sid = lax.axis_index("s")
    wid = sid * NC + cid

    def _zero_rows(j, _):
        for i in range(DIM // 16):
            rows_a[j, pl.ds(i * 16, 16)] = jnp.zeros((16,), jnp.float32)
        return 0

    lax.fori_loop(0, EC, _zero_rows, 0)
    for r in range(ROWS_PER_SUB // EC):
        pltpu.sync_copy(rows_a, agg.at[pl.ds(sid * ROWS_PER_SUB + r * EC, EC)])
    plsc.subcore_barrier()

    def _chunk(t, _):
        eb = (wid * CHUNKS_PER_W + t) * EC
        pltpu.sync_copy(src_hbm.at[pl.ds(eb, EC)], idx_s)
        pltpu.sync_copy(dst_hbm.at[pl.ds(eb, EC)], idx_d)
        pltpu.async_copy(ps_hbm.at[idx_s], rows_a, sem).wait()
        pltpu.async_copy(pd_hbm.at[idx_d], rows_b, sem).wait()

        def _relu_add(j, _):
            for i in range(DIM // 16):
                a = rows_a[j, pl.ds(i * 16, 16)]
                b = rows_b[j, pl.ds(i * 16, 16)]
                rows_a[j, pl.ds(i * 16, 16)] = jnp.maximum(a + b, 0.0)
            return 0

        lax.fori_loop(0, EC, _relu_add, 0)
        pltpu.sync_copy(rows_a, agg.at[idx_d], add=True)
        return 0

    lax.fori_loop(0, CHUNKS_PER_W, _chunk, 0)
    plsc.subcore_barrier()

    for r in range(ROWS_PER_SUB // EC):
        off = sid * ROWS_PER_SUB + r * EC
        pltpu.sync_copy(agg.at[pl.ds(off, EC)],
                        out_hbm.at[pl.ds(cid * N_PAD + off, EC)])


def _sc_edge_stage(ps, pd, src_p, dst_p):
    mesh = plsc.VectorSubcoreMesh(
        core_axis_name="c", subcore_axis_name="s", num_cores=NC, num_subcores=NS
    )
    return pl.kernel(
        _sc_body,
        out_type=jax.ShapeDtypeStruct((NC * N_PAD, DIM), jnp.float32),
        mesh=mesh,
        scratch_types=[
            pltpu.VMEM((EC,), jnp.int32),
            pltpu.VMEM((EC,), jnp.int32),
            pltpu.VMEM((EC, DIM), jnp.float32),
            pltpu.VMEM((EC, DIM), jnp.float32),
            pltpu.VMEM_SHARED((N_PAD, DIM), jnp.float32),
            pltpu.SemaphoreType.DMA,
        ],
    )(ps, pd, src_p, dst_p)


def _final_body(x_ref, agg_ref, wn_ref, bn_ref, o_ref):
    a = agg_ref[0:N_NODES, :] + agg_ref[N_PAD : N_PAD + N_NODES, :]
    o_ref[...] = (
        jnp.dot(x_ref[...], wn_ref[0:DIM, :], preferred_element_type=jnp.float32)
        + jnp.dot(a, wn_ref[DIM : 2 * DIM, :], preferred_element_type=jnp.float32)
        + bn_ref[...]
    )


def _final(x, agg2, W_node, b_node):
    return pl.pallas_call(
        _final_body,
        out_shape=jax.ShapeDtypeStruct((N_NODES, DIM), jnp.float32),
    )(x, agg2, W_node, b_node.reshape(1, DIM))


def kernel(x, edge_index, W_edge, b_edge, W_node, b_node):
    src = edge_index[0].astype(jnp.int32)
    dst = edge_index[1].astype(jnp.int32)
    n_dummy = E_PAD - N_EDGES
    dummy = jnp.full((n_dummy,), N_NODES, dtype=jnp.int32)
    src_p = jnp.concatenate([src, dummy])
    dst_p = jnp.concatenate([dst, dummy])
    x_pad = jnp.pad(x, ((0, N_PAD - N_NODES), (0, 0)))

    ps, pd = _project(x_pad, W_edge, b_edge)
    agg2 = _sc_edge_stage(ps, pd, src_p, dst_p)
    return _final(x, agg2, W_node, b_node)


# speedup vs baseline: 8.9114x; 2.5083x over previous
import jax
import jax.numpy as jnp
from jax import lax
from jax.experimental import pallas as pl
from jax.experimental.pallas import tpu as pltpu
from jax.experimental.pallas import tpu_sc as plsc

N_NODES = 10000
N_EDGES = 320000
DIM = 128

NC = 2
NS = 16
NW = NC * NS

N_PAD = 10240
EC = 80
CHUNKS_PER_W = 125
ROWS_PER_SUB = N_PAD // NS


def _proj_body(x_ref, we_ref, be_ref, ps_ref, pd_ref):
    xv = x_ref[...]
    ps_ref[...] = jnp.dot(xv, we_ref[0:DIM, :], preferred_element_type=jnp.float32)
    pd_ref[...] = (
        jnp.dot(xv, we_ref[DIM : 2 * DIM, :], preferred_element_type=jnp.float32)
        + be_ref[...]
    )


def _project(x, W_edge, b_edge):
    return pl.pallas_call(
        _proj_body,
        out_shape=[
            jax.ShapeDtypeStruct((N_NODES, DIM), jnp.float32),
            jax.ShapeDtypeStruct((N_NODES, DIM), jnp.float32),
        ],
    )(x, W_edge, b_edge.reshape(1, DIM))


def _sc_body(ps_hbm, pd_hbm, src_hbm, dst_hbm, out_hbm,
             idx_s, idx_d, rows_a, rows_b, agg, sem0, sem1):
    cid = lax.axis_index("c")
    sid = lax.axis_index("s")
    wid = sid * NC + cid
    sems = (sem0, sem1)

    @pl.loop(0, EC)
    def _zero_rows(j):
        for i in range(DIM // 16):
            rows_a[0, j, pl.ds(i * 16, 16)] = jnp.zeros((16,), jnp.float32)

    for r in range(ROWS_PER_SUB // EC):
        pltpu.sync_copy(rows_a.at[0],
                        agg.at[pl.ds(sid * ROWS_PER_SUB + r * EC, EC)])
    plsc.subcore_barrier()

    def _load_idx(t, p):
        eb = (wid * CHUNKS_PER_W + t) * EC
        pltpu.sync_copy(src_hbm.at[pl.ds(eb, EC)], idx_s.at[p])
        pltpu.sync_copy(dst_hbm.at[pl.ds(eb, EC)], idx_d.at[p])

    def _issue_gathers(p):
        pltpu.async_copy(ps_hbm.at[idx_s.at[p]], rows_a.at[p], sems[p])
        pltpu.async_copy(pd_hbm.at[idx_d.at[p]], rows_b.at[p], sems[p])

    def _wait_gathers(p):
        pltpu.make_async_copy(ps_hbm.at[idx_s.at[p]], rows_a.at[p], sems[p]).wait()
        pltpu.make_async_copy(pd_hbm.at[idx_d.at[p]], rows_b.at[p], sems[p]).wait()

    def _process(t, p, prefetch):
        _wait_gathers(p)

        @plsc.parallel_loop(0, EC, unroll=4)
        def _relu_add(j):
            for i in range(DIM // 16):
                a = rows_a[p, j, pl.ds(i * 16, 16)]
                b = rows_b[p, j, pl.ds(i * 16, 16)]
                rows_a[p, j, pl.ds(i * 16, 16)] = jnp.maximum(a + b, 0.0)

        pltpu.sync_copy(rows_a.at[p], agg.at[idx_d.at[p]], add=True)
        if prefetch:
            _load_idx(t + 2, p)
            _issue_gathers(p)

    _load_idx(0, 0)
    _issue_gathers(0)
    _load_idx(1, 1)
    _issue_gathers(1)

    @pl.loop(0, CHUNKS_PER_W - 3, step=2)
    def _pair(t):
        _process(t, 0, True)
        _process(t + 1, 1, True)

    _process(CHUNKS_PER_W - 3, 0, True)
    _process(CHUNKS_PER_W - 2, 1, False)
    _process(CHUNKS_PER_W - 1, 0, False)
    plsc.subcore_barrier()

    for r in range(ROWS_PER_SUB // EC):
        off = sid * ROWS_PER_SUB + r * EC
        pltpu.sync_copy(agg.at[pl.ds(off, EC)],
                        out_hbm.at[pl.ds(cid * N_PAD + off, EC)])


def _sc_edge_stage(ps, pd, src_p, dst_p):
    mesh = plsc.VectorSubcoreMesh(
        core_axis_name="c", subcore_axis_name="s", num_cores=NC, num_subcores=NS
    )
    return pl.kernel(
        _sc_body,
        out_type=jax.ShapeDtypeStruct((NC * N_PAD, DIM), jnp.float32),
        mesh=mesh,
        scratch_types=[
            pltpu.VMEM((2, EC), jnp.int32),
            pltpu.VMEM((2, EC), jnp.int32),
            pltpu.VMEM((2, EC, DIM), jnp.float32),
            pltpu.VMEM((2, EC, DIM), jnp.float32),
            pltpu.VMEM_SHARED((N_PAD, DIM), jnp.float32),
            pltpu.SemaphoreType.DMA,
            pltpu.SemaphoreType.DMA,
        ],
    )(ps, pd, src_p, dst_p)


def _final_body(x_ref, agg_ref, wn_ref, bn_ref, o_ref):
    a = agg_ref[0:N_NODES, :] + agg_ref[N_PAD : N_PAD + N_NODES, :]
    o_ref[...] = (
        jnp.dot(x_ref[...], wn_ref[0:DIM, :], preferred_element_type=jnp.float32)
        + jnp.dot(a, wn_ref[DIM : 2 * DIM, :], preferred_element_type=jnp.float32)
        + bn_ref[...]
    )


def _final(x, agg2, W_node, b_node):
    return pl.pallas_call(
        _final_body,
        out_shape=jax.ShapeDtypeStruct((N_NODES, DIM), jnp.float32),
    )(x, agg2, W_node, b_node.reshape(1, DIM))


def kernel(x, edge_index, W_edge, b_edge, W_node, b_node):
    src = edge_index[0].astype(jnp.int32)
    dst = edge_index[1].astype(jnp.int32)
    ps, pd = _project(x, W_edge, b_edge)
    agg2 = _sc_edge_stage(ps, pd, src, dst)
    return _final(x, agg2, W_node, b_node)


# speedup vs baseline: 11.7718x; 1.3210x over previous
import jax
import jax.numpy as jnp
from jax import lax
from jax.experimental import pallas as pl
from jax.experimental.pallas import tpu as pltpu
from jax.experimental.pallas import tpu_sc as plsc

N_NODES = 10000
N_EDGES = 320000
DIM = 128

NC = 2
NS = 16
NW = NC * NS

EC = 64
CHUNKS_PER_W = 157
E_PAD = NW * CHUNKS_PER_W * EC
N_TAB = 10112
ROWS_PER_SUB = N_TAB // NS


def _proj_body(x_ref, we_ref, be_ref, ps_ref, pd_ref):
    xv = x_ref[...]
    ps_ref[...] = jnp.dot(xv, we_ref[0:DIM, :], preferred_element_type=jnp.float32)
    pd_ref[...] = (
        jnp.dot(xv, we_ref[DIM : 2 * DIM, :], preferred_element_type=jnp.float32)
        + be_ref[...]
    )


def _project(x_pad, W_edge, b_edge):
    return pl.pallas_call(
        _proj_body,
        out_shape=[
            jax.ShapeDtypeStruct((N_TAB, DIM), jnp.float32),
            jax.ShapeDtypeStruct((N_TAB, DIM), jnp.float32),
        ],
    )(x_pad, W_edge, b_edge.reshape(1, DIM))


def _sc_body(ps_hbm, pd_hbm, src_hbm, dst_hbm, out_hbm,
             idx_s, idx_d, rows_a, rows_b, rows_m, agg,
             gsem0, gsem1, ssem0, ssem1, isem):
    cid = lax.axis_index("c")
    sid = lax.axis_index("s")
    wid = sid * NC + cid
    gsems = (gsem0, gsem1)
    ssems = (ssem0, ssem1)

    @pl.loop(0, EC)
    def _zero_rows(j):
        for i in range(DIM // 16):
            rows_a[0, j, pl.ds(i * 16, 16)] = jnp.zeros((16,), jnp.float32)

    row0 = sid * ROWS_PER_SUB
    for r in range(ROWS_PER_SUB // EC):
        pltpu.sync_copy(rows_a.at[0], agg.at[pl.ds(row0 + r * EC, EC)])
    rem = ROWS_PER_SUB % EC
    if rem:
        pltpu.sync_copy(rows_a.at[0, pl.ds(0, rem)],
                        agg.at[pl.ds(row0 + ROWS_PER_SUB - rem, rem)])
    plsc.subcore_barrier()

    def _issue_idx(t, q):
        eb = (wid * CHUNKS_PER_W + t) * EC
        pltpu.async_copy(src_hbm.at[pl.ds(eb, EC)], idx_s.at[q], isem)
        pltpu.async_copy(dst_hbm.at[pl.ds(eb, EC)], idx_d.at[q], isem)

    def _wait_idx(q):
        pltpu.make_async_copy(src_hbm.at[pl.ds(0, EC)], idx_s.at[q], isem).wait()
        pltpu.make_async_copy(dst_hbm.at[pl.ds(0, EC)], idx_d.at[q], isem).wait()

    def _issue_gathers(p, q):
        pltpu.async_copy(ps_hbm.at[idx_s.at[q]], rows_a.at[p], gsems[p])
        pltpu.async_copy(pd_hbm.at[idx_d.at[q]], rows_b.at[p], gsems[p])

    def _wait_gathers(p, q):
        pltpu.make_async_copy(ps_hbm.at[idx_s.at[q]], rows_a.at[p], gsems[p]).wait()
        pltpu.make_async_copy(pd_hbm.at[idx_d.at[q]], rows_b.at[p], gsems[p]).wait()

    def _issue_scatter(p, q):
        pltpu.async_copy(rows_m.at[p], agg.at[idx_d.at[q]], ssems[p], add=True)

    def _wait_scatter(p, q):
        pltpu.make_async_copy(rows_m.at[p], agg.at[idx_d.at[q]], ssems[p]).wait()

    def _compute(p):
        @plsc.parallel_loop(0, EC, unroll=4)
        def _relu_add(j):
            for i in range(DIM // 16):
                a = rows_a[p, j, pl.ds(i * 16, 16)]
                b = rows_b[p, j, pl.ds(i * 16, 16)]
                rows_m[p, j, pl.ds(i * 16, 16)] = jnp.maximum(a + b, 0.0)

    def _process(t, p, first=False, last=False):
        q = t % 4
        qn = (t + 2) % 4
        _wait_gathers(p, q)
        if not first:
            _wait_scatter(p, qn)
        if not last:
            _issue_idx(t + 2, qn)
        _compute(p)
        _issue_scatter(p, q)
        if not last:
            _wait_idx(qn)
            _issue_gathers(p, qn)

    _issue_idx(0, 0)
    _issue_idx(1, 1)
    _wait_idx(0)
    _issue_gathers(0, 0)
    _wait_idx(1)
    _issue_gathers(1, 1)
    _process(0, 0, first=True)
    _process(1, 1, first=True)

    @pl.loop(2, CHUNKS_PER_W - 3, step=2)
    def _pair(t):
        _process(t, 0)
        _process(t + 1, 1)

    _process(CHUNKS_PER_W - 3, 0)
    _process(CHUNKS_PER_W - 2, 1, last=True)
    _process(CHUNKS_PER_W - 1, 0, last=True)
    _wait_scatter((CHUNKS_PER_W - 2) % 2, (CHUNKS_PER_W - 2) % 4)
    _wait_scatter((CHUNKS_PER_W - 1) % 2, (CHUNKS_PER_W - 1) % 4)
    plsc.subcore_barrier()

    for r in range(ROWS_PER_SUB // EC):
        off = row0 + r * EC
        pltpu.sync_copy(agg.at[pl.ds(off, EC)],
                        out_hbm.at[pl.ds(cid * N_TAB + off, EC)])
    if rem:
        off = row0 + ROWS_PER_SUB - rem
        pltpu.sync_copy(agg.at[pl.ds(off, rem)],
                        out_hbm.at[pl.ds(cid * N_TAB + off, rem)])


def _sc_edge_stage(ps, pd, src_p, dst_p):
    mesh = plsc.VectorSubcoreMesh(
        core_axis_name="c", subcore_axis_name="s", num_cores=NC, num_subcores=NS
    )
    return pl.kernel(
        _sc_body,
        out_type=jax.ShapeDtypeStruct((NC * N_TAB, DIM), jnp.float32),
        mesh=mesh,
        scratch_types=[
            pltpu.VMEM((4, EC), jnp.int32),
            pltpu.VMEM((4, EC), jnp.int32),
            pltpu.VMEM((2, EC, DIM), jnp.float32),
            pltpu.VMEM((2, EC, DIM), jnp.float32),
            pltpu.VMEM((2, EC, DIM), jnp.float32),
            pltpu.VMEM_SHARED((N_TAB, DIM), jnp.float32),
            pltpu.SemaphoreType.DMA,
            pltpu.SemaphoreType.DMA,
            pltpu.SemaphoreType.DMA,
            pltpu.SemaphoreType.DMA,
            pltpu.SemaphoreType.DMA,
        ],
    )(ps, pd, src_p, dst_p)


def _final_body(x_ref, agg_ref, wn_ref, bn_ref, o_ref):
    a = agg_ref[0:N_NODES, :] + agg_ref[N_TAB : N_TAB + N_NODES, :]
    o_ref[...] = (
        jnp.dot(x_ref[...], wn_ref[0:DIM, :], preferred_element_type=jnp.float32)
        + jnp.dot(a, wn_ref[DIM : 2 * DIM, :], preferred_element_type=jnp.float32)
        + bn_ref[...]
    )


def _final(x, agg2, W_node, b_node):
    return pl.pallas_call(
        _final_body,
        out_shape=jax.ShapeDtypeStruct((N_NODES, DIM), jnp.float32),
    )(x, agg2, W_node, b_node.reshape(1, DIM))


def kernel(x, edge_index, W_edge, b_edge, W_node, b_node):
    src = edge_index[0].astype(jnp.int32)
    dst = edge_index[1].astype(jnp.int32)
    n_dummy = E_PAD - N_EDGES
    ar = jnp.arange(n_dummy, dtype=jnp.int32)
    src_p = jnp.concatenate([src, ar % N_NODES])
    dst_p = jnp.concatenate([dst, N_NODES + ar % (N_TAB - N_NODES)])
    x_pad = jnp.pad(x, ((0, N_TAB - N_NODES), (0, 0)))

    ps, pd = _project(x_pad, W_edge, b_edge)
    agg2 = _sc_edge_stage(ps, pd, src_p, dst_p)
    return _final(x, agg2, W_node, b_node)
